# baseline (speedup 1.0000x reference)
import functools

import jax
import jax.numpy as jnp
from jax import lax
from jax.experimental import pallas as pl
from jax.experimental.pallas import tpu as pltpu
from jax.experimental.pallas import tpu_sc as plsc

L = 16
NC = 2
NS = 16
NW = NC * NS
CH = 128
NBUF = 10
TH = 1024


def _combined_tc(type_table, pos_embedding):
    T, D = type_table.shape
    S = pos_embedding.shape[1]

    def body(type_ref, pos_ref, out_ref):
        t = type_ref[...]
        p = pos_ref[...]
        out_ref[...] = t[:, None, :] + p[0][None, :, :]

    out = pl.pallas_call(
        body,
        out_shape=jax.ShapeDtypeStruct((T, S, D), jnp.float32),
    )(type_table, pos_embedding)
    return out.reshape(T * S, D)


def _sc_lookup(tok_flat, tt_flat, pos_flat, word_table, combined, seq_len):
    N = tok_flat.shape[0]
    D = word_table.shape[1]
    S = seq_len
    per_w = N // NW
    n_ch = per_w // CH
    n_grp = n_ch // NBUF
    assert per_w % CH == 0 and n_ch % NBUF == 0

    mesh = plsc.VectorSubcoreMesh(core_axis_name="c", subcore_axis_name="s")

    @functools.partial(
        pl.kernel,
        out_type=jax.ShapeDtypeStruct((N, D), jnp.float32),
        mesh=mesh,
        compiler_params=pltpu.CompilerParams(use_tc_tiling_on_sc=False),
        scratch_types=[
            pltpu.VMEM((NBUF, CH), jnp.int32),
            pltpu.VMEM((NBUF, CH), jnp.int32),
            pltpu.VMEM((NBUF, CH), jnp.int32),
            pltpu.VMEM((NBUF, CH), jnp.int32),
            pltpu.VMEM((NBUF, CH, D), jnp.float32),
            pltpu.SemaphoreType.DMA,
            pltpu.SemaphoreType.DMA,
            pltpu.SemaphoreType.DMA,
            pltpu.SemaphoreType.DMA,
        ],
    )
    def sc_k(tok_hbm, tt_hbm, pos_hbm, word_hbm, comb_hbm, out_hbm,
             tok_v, tt_v, pos_v, cidx_v, rows_v, sem_i, sem_g, sem_a, sem_w):
        wid = lax.axis_index("s") * NC + lax.axis_index("c")
        base = wid * per_w

        def fire_i(goff, s):
            off = goff + s * CH
            pltpu.async_copy(tok_hbm.at[pl.ds(off, CH)], tok_v.at[s], sem_i)
            pltpu.async_copy(tt_hbm.at[pl.ds(off, CH)], tt_v.at[s], sem_i)
            pltpu.async_copy(pos_hbm.at[pl.ds(off, CH)], pos_v.at[s], sem_i)

        def drain_i(goff, s):
            off = goff + s * CH
            pltpu.make_async_copy(tok_hbm.at[pl.ds(off, CH)], tok_v.at[s], sem_i).wait()
            pltpu.make_async_copy(tt_hbm.at[pl.ds(off, CH)], tt_v.at[s], sem_i).wait()
            pltpu.make_async_copy(pos_hbm.at[pl.ds(off, CH)], pos_v.at[s], sem_i).wait()

        for s in range(NBUF):
            fire_i(base, s)

        def group(g, carry):
            goff = base + g * (NBUF * CH)
            for s in range(NBUF):
                drain_i(goff, s)
                for k in range(CH // L):
                    sl = pl.ds(k * L, L)
                    v = tok_v[s, sl]
                    j = v & (2 * TH - 1)
                    tok_v[s, sl] = (v - j) + 2 * j - jnp.where(j < TH, 0, 2 * TH - 1)
                pltpu.async_copy(word_hbm.at[tok_v.at[s]], rows_v.at[s], sem_g)
            for s in range(NBUF):
                for k in range(CH // L):
                    sl = pl.ds(k * L, L)
                    cidx_v[s, sl] = tt_v[s, sl] * S + pos_v[s, sl]
            for s in range(NBUF):
                pltpu.make_async_copy(word_hbm.at[tok_v.at[s]], rows_v.at[s], sem_g).wait()
            for s in range(NBUF):
                pltpu.async_copy(comb_hbm.at[cidx_v.at[s]], rows_v.at[s], sem_a, add=True)
            for s in range(NBUF):
                pltpu.make_async_copy(comb_hbm.at[cidx_v.at[s]], rows_v.at[s], sem_a).wait()
            for s in range(NBUF):
                off = goff + s * CH
                pltpu.async_copy(rows_v.at[s], out_hbm.at[pl.ds(off, CH)], sem_w)
            @pl.when(g + 1 < n_grp)
            def _():
                for s in range(NBUF):
                    fire_i(goff + NBUF * CH, s)
            for s in range(NBUF):
                off = goff + s * CH
                pltpu.make_async_copy(rows_v.at[s], out_hbm.at[pl.ds(off, CH)], sem_w).wait()
            return carry

        lax.fori_loop(0, n_grp, group, 0)

    return sc_k(tok_flat, tt_flat, pos_flat, word_table, combined)


def _transpose_table_tc(word_table):
    D, V = word_table.T.shape
    wt_T = word_table.T
    H = TH
    VB = 2 * H
    grid = pl.cdiv(V, VB)

    def body(eye_ref, in_ref, out_ref):
        t = jax.lax.dot_general(
            in_ref[...], eye_ref[...],
            dimension_numbers=(((0,), (0,)), ((), ())),
            preferred_element_type=jnp.float32)
        out_ref[...] = jnp.concatenate([t[:H], t[H:]], axis=1)

    eye = jnp.eye(D, dtype=jnp.float32)
    return pl.pallas_call(
        body,
        grid=(grid,),
        in_specs=[pl.BlockSpec((D, D), lambda i: (0, 0)),
                  pl.BlockSpec((D, VB), lambda i: (0, i))],
        out_specs=pl.BlockSpec((H, 2 * D), lambda i: (i, 0)),
        out_shape=jax.ShapeDtypeStruct((grid * H, 2 * D), jnp.float32),
    )(eye, wt_T)


def kernel(tokens, token_types, word_table, type_table, pos_embedding):
    B, S = tokens.shape
    D = word_table.shape[1]
    V = word_table.shape[0]
    tok_flat = tokens.reshape(-1).astype(jnp.int32)
    tt_flat = token_types.reshape(-1).astype(jnp.int32)
    pos_flat = jnp.broadcast_to(
        jnp.arange(S, dtype=jnp.int32)[None, :], (B, S)).reshape(-1)
    combined = _combined_tc(type_table.astype(jnp.float32),
                            pos_embedding.astype(jnp.float32))
    wt_pairs = _transpose_table_tc(word_table)
    wt_rows = wt_pairs.reshape(wt_pairs.shape[0] * 2, D)
    out = _sc_lookup(tok_flat, tt_flat, pos_flat, wt_rows, combined, S)
    return out.reshape(B, S, D)


# speedup vs baseline: 1.3612x; 1.3612x over previous
import functools

import jax
import jax.numpy as jnp
from jax import lax
from jax.experimental import pallas as pl
from jax.experimental.pallas import tpu as pltpu
from jax.experimental.pallas import tpu_sc as plsc

L = 16
NC = 2
NS = 16
NW = NC * NS
CH = 128
NBUF = 10
TH = 4096


def _combined_tc(type_table, pos_embedding):
    T, D = type_table.shape
    S = pos_embedding.shape[1]

    def body(type_ref, pos_ref, out_ref):
        t = type_ref[...]
        p = pos_ref[...]
        out_ref[...] = t[:, None, :] + p[0][None, :, :]

    out = pl.pallas_call(
        body,
        out_shape=jax.ShapeDtypeStruct((T, S, D), jnp.float32),
    )(type_table, pos_embedding)
    return out.reshape(T * S, D)


def _sc_lookup(tok_flat, tt_flat, pos_flat, word_table, combined, seq_len):
    N = tok_flat.shape[0]
    D = word_table.shape[1]
    S = seq_len
    per_w = N // NW
    n_ch = per_w // CH
    n_grp = n_ch // NBUF
    assert per_w % CH == 0 and n_ch % NBUF == 0

    mesh = plsc.VectorSubcoreMesh(core_axis_name="c", subcore_axis_name="s")

    @functools.partial(
        pl.kernel,
        out_type=jax.ShapeDtypeStruct((N, D), jnp.float32),
        mesh=mesh,
        compiler_params=pltpu.CompilerParams(use_tc_tiling_on_sc=False),
        scratch_types=[
            pltpu.VMEM((NBUF, CH), jnp.int32),
            pltpu.VMEM((NBUF, CH), jnp.int32),
            pltpu.VMEM((NBUF, CH), jnp.int32),
            pltpu.VMEM((NBUF, CH), jnp.int32),
            pltpu.VMEM((NBUF, CH, D), jnp.float32),
            pltpu.SemaphoreType.DMA,
            pltpu.SemaphoreType.DMA,
            pltpu.SemaphoreType.DMA,
            pltpu.SemaphoreType.DMA,
        ],
    )
    def sc_k(tok_hbm, tt_hbm, pos_hbm, word_hbm, comb_hbm, out_hbm,
             tok_v, tt_v, pos_v, cidx_v, rows_v, sem_i, sem_g, sem_a, sem_w):
        wid = lax.axis_index("s") * NC + lax.axis_index("c")
        base = wid * per_w

        def fire_i(goff, s):
            off = goff + s * CH
            pltpu.async_copy(tok_hbm.at[pl.ds(off, CH)], tok_v.at[s], sem_i)
            pltpu.async_copy(tt_hbm.at[pl.ds(off, CH)], tt_v.at[s], sem_i)
            pltpu.async_copy(pos_hbm.at[pl.ds(off, CH)], pos_v.at[s], sem_i)

        def drain_i(goff, s):
            off = goff + s * CH
            pltpu.make_async_copy(tok_hbm.at[pl.ds(off, CH)], tok_v.at[s], sem_i).wait()
            pltpu.make_async_copy(tt_hbm.at[pl.ds(off, CH)], tt_v.at[s], sem_i).wait()
            pltpu.make_async_copy(pos_hbm.at[pl.ds(off, CH)], pos_v.at[s], sem_i).wait()

        for s in range(NBUF):
            fire_i(base, s)

        def group(g, carry):
            goff = base + g * (NBUF * CH)
            for s in range(NBUF):
                drain_i(goff, s)
                for k in range(CH // L):
                    sl = pl.ds(k * L, L)
                    v = tok_v[s, sl]
                    j = v & (2 * TH - 1)
                    tok_v[s, sl] = (v - j) + 2 * j - jnp.where(j < TH, 0, 2 * TH - 1)
                pltpu.async_copy(word_hbm.at[tok_v.at[s]], rows_v.at[s], sem_g)
            for s in range(NBUF):
                for k in range(CH // L):
                    sl = pl.ds(k * L, L)
                    cidx_v[s, sl] = tt_v[s, sl] * S + pos_v[s, sl]
            for s in range(NBUF):
                pltpu.make_async_copy(word_hbm.at[tok_v.at[s]], rows_v.at[s], sem_g).wait()
            for s in range(NBUF):
                pltpu.async_copy(comb_hbm.at[cidx_v.at[s]], rows_v.at[s], sem_a, add=True)
            for s in range(NBUF):
                pltpu.make_async_copy(comb_hbm.at[cidx_v.at[s]], rows_v.at[s], sem_a).wait()
            for s in range(NBUF):
                off = goff + s * CH
                pltpu.async_copy(rows_v.at[s], out_hbm.at[pl.ds(off, CH)], sem_w)
            @pl.when(g + 1 < n_grp)
            def _():
                for s in range(NBUF):
                    fire_i(goff + NBUF * CH, s)
            for s in range(NBUF):
                off = goff + s * CH
                pltpu.make_async_copy(rows_v.at[s], out_hbm.at[pl.ds(off, CH)], sem_w).wait()
            return carry

        lax.fori_loop(0, n_grp, group, 0)

    return sc_k(tok_flat, tt_flat, pos_flat, word_table, combined)


def _transpose_table_tc(word_table):
    D, V = word_table.T.shape
    wt_T = word_table.T
    H = TH
    VB = 2 * H
    grid = pl.cdiv(V, VB)

    def body(in_ref, out_ref):
        t = in_ref[...].T
        out_ref[...] = jnp.concatenate([t[:H], t[H:]], axis=1)

    return pl.pallas_call(
        body,
        grid=(grid,),
        in_specs=[pl.BlockSpec((D, VB), lambda i: (0, i))],
        out_specs=pl.BlockSpec((H, 2 * D), lambda i: (i, 0)),
        out_shape=jax.ShapeDtypeStruct((grid * H, 2 * D), jnp.float32),
    )(wt_T)


def kernel(tokens, token_types, word_table, type_table, pos_embedding):
    B, S = tokens.shape
    D = word_table.shape[1]
    V = word_table.shape[0]
    tok_flat = tokens.reshape(-1).astype(jnp.int32)
    tt_flat = token_types.reshape(-1).astype(jnp.int32)
    pos_flat = jnp.broadcast_to(
        jnp.arange(S, dtype=jnp.int32)[None, :], (B, S)).reshape(-1)
    combined = _combined_tc(type_table.astype(jnp.float32),
                            pos_embedding.astype(jnp.float32))
    wt_pairs = _transpose_table_tc(word_table)
    wt_rows = wt_pairs.reshape(wt_pairs.shape[0] * 2, D)
    out = _sc_lookup(tok_flat, tt_flat, pos_flat, wt_rows, combined, S)
    return out.reshape(B, S, D)


# speedup vs baseline: 1.4467x; 1.0628x over previous
import functools

import jax
import jax.numpy as jnp
from jax import lax
from jax.experimental import pallas as pl
from jax.experimental.pallas import tpu as pltpu
from jax.experimental.pallas import tpu_sc as plsc

L = 16
NC = 2
NS = 16
NW = NC * NS
CH = 128
NBUF = 10
TH = 8192


def _combined_tc(type_table, pos_embedding):
    T, D = type_table.shape
    S = pos_embedding.shape[1]

    def body(type_ref, pos_ref, out_ref):
        t = type_ref[...]
        p = pos_ref[...]
        out_ref[...] = t[:, None, :] + p[0][None, :, :]

    out = pl.pallas_call(
        body,
        out_shape=jax.ShapeDtypeStruct((T, S, D), jnp.float32),
    )(type_table, pos_embedding)
    return out.reshape(T * S, D)


def _sc_lookup(tok_flat, tt_flat, pos_flat, word_table, combined, seq_len):
    N = tok_flat.shape[0]
    D = word_table.shape[1]
    S = seq_len
    per_w = N // NW
    n_ch = per_w // CH
    n_grp = n_ch // NBUF
    assert per_w % CH == 0 and n_ch % NBUF == 0

    mesh = plsc.VectorSubcoreMesh(core_axis_name="c", subcore_axis_name="s")

    @functools.partial(
        pl.kernel,
        out_type=jax.ShapeDtypeStruct((N, D), jnp.float32),
        mesh=mesh,
        compiler_params=pltpu.CompilerParams(use_tc_tiling_on_sc=False),
        scratch_types=[
            pltpu.VMEM((NBUF, CH), jnp.int32),
            pltpu.VMEM((NBUF, CH), jnp.int32),
            pltpu.VMEM((NBUF, CH), jnp.int32),
            pltpu.VMEM((NBUF, CH), jnp.int32),
            pltpu.VMEM((NBUF, CH, D), jnp.float32),
            pltpu.SemaphoreType.DMA,
            pltpu.SemaphoreType.DMA,
            pltpu.SemaphoreType.DMA,
            pltpu.SemaphoreType.DMA,
        ],
    )
    def sc_k(tok_hbm, tt_hbm, pos_hbm, word_hbm, comb_hbm, out_hbm,
             tok_v, tt_v, pos_v, cidx_v, rows_v, sem_i, sem_g, sem_a, sem_w):
        wid = lax.axis_index("s") * NC + lax.axis_index("c")
        base = wid * per_w

        def fire_i(goff, s):
            off = goff + s * CH
            pltpu.async_copy(tok_hbm.at[pl.ds(off, CH)], tok_v.at[s], sem_i)
            pltpu.async_copy(tt_hbm.at[pl.ds(off, CH)], tt_v.at[s], sem_i)
            pltpu.async_copy(pos_hbm.at[pl.ds(off, CH)], pos_v.at[s], sem_i)

        def drain_i(goff, s):
            off = goff + s * CH
            pltpu.make_async_copy(tok_hbm.at[pl.ds(off, CH)], tok_v.at[s], sem_i).wait()
            pltpu.make_async_copy(tt_hbm.at[pl.ds(off, CH)], tt_v.at[s], sem_i).wait()
            pltpu.make_async_copy(pos_hbm.at[pl.ds(off, CH)], pos_v.at[s], sem_i).wait()

        for s in range(NBUF):
            fire_i(base, s)

        def group(g, carry):
            goff = base + g * (NBUF * CH)
            for s in range(NBUF):
                drain_i(goff, s)
                for k in range(CH // L):
                    sl = pl.ds(k * L, L)
                    v = tok_v[s, sl]
                    j = v & (2 * TH - 1)
                    tok_v[s, sl] = (v - j) + 2 * j - jnp.where(j < TH, 0, 2 * TH - 1)
                pltpu.async_copy(word_hbm.at[tok_v.at[s]], rows_v.at[s], sem_g)
            for s in range(NBUF):
                for k in range(CH // L):
                    sl = pl.ds(k * L, L)
                    cidx_v[s, sl] = tt_v[s, sl] * S + pos_v[s, sl]
            for s in range(NBUF):
                pltpu.make_async_copy(word_hbm.at[tok_v.at[s]], rows_v.at[s], sem_g).wait()
            for s in range(NBUF):
                pltpu.async_copy(comb_hbm.at[cidx_v.at[s]], rows_v.at[s], sem_a, add=True)
            for s in range(NBUF):
                pltpu.make_async_copy(comb_hbm.at[cidx_v.at[s]], rows_v.at[s], sem_a).wait()
            for s in range(NBUF):
                off = goff + s * CH
                pltpu.async_copy(rows_v.at[s], out_hbm.at[pl.ds(off, CH)], sem_w)
            @pl.when(g + 1 < n_grp)
            def _():
                for s in range(NBUF):
                    fire_i(goff + NBUF * CH, s)
            for s in range(NBUF):
                off = goff + s * CH
                pltpu.make_async_copy(rows_v.at[s], out_hbm.at[pl.ds(off, CH)], sem_w).wait()
            return carry

        lax.fori_loop(0, n_grp, group, 0)

    return sc_k(tok_flat, tt_flat, pos_flat, word_table, combined)


def _transpose_table_tc(word_table):
    D, V = word_table.T.shape
    wt_T = word_table.T
    H = TH
    VB = 2 * H
    grid = pl.cdiv(V, VB)

    def body(in_ref, out_ref):
        t = in_ref[...].T
        out_ref[...] = jnp.concatenate([t[:H], t[H:]], axis=1)

    return pl.pallas_call(
        body,
        grid=(grid,),
        in_specs=[pl.BlockSpec((D, VB), lambda i: (0, i))],
        out_specs=pl.BlockSpec((H, 2 * D), lambda i: (i, 0)),
        out_shape=jax.ShapeDtypeStruct((grid * H, 2 * D), jnp.float32),
    )(wt_T)


def kernel(tokens, token_types, word_table, type_table, pos_embedding):
    B, S = tokens.shape
    D = word_table.shape[1]
    V = word_table.shape[0]
    tok_flat = tokens.reshape(-1).astype(jnp.int32)
    tt_flat = token_types.reshape(-1).astype(jnp.int32)
    pos_flat = jnp.broadcast_to(
        jnp.arange(S, dtype=jnp.int32)[None, :], (B, S)).reshape(-1)
    combined = _combined_tc(type_table.astype(jnp.float32),
                            pos_embedding.astype(jnp.float32))
    wt_pairs = _transpose_table_tc(word_table)
    wt_rows = wt_pairs.reshape(wt_pairs.shape[0] * 2, D)
    out = _sc_lookup(tok_flat, tt_flat, pos_flat, wt_rows, combined, S)
    return out.reshape(B, S, D)
